# Initial kernel scaffold; baseline (speedup 1.0000x reference)
import jax
import jax.numpy as jnp
from jax.experimental import pallas as pl


def kernel(x, edge_index, W, b):
    raise NotImplementedError("write your pallas kernel here")


# speedup vs baseline: 77.3571x; 77.3571x over previous
import functools

import jax
import jax.numpy as jnp
from jax import lax
from jax.experimental import pallas as pl
from jax.experimental.pallas import tpu as pltpu
from jax.experimental.pallas import tpu_sc as plsc

N_NODES = 10000
N_EDGES = 160000
D_IN = 256
D_OUT = 512

NC = 2
NS = 16
LANES = 16

NN = 10240
SL = NN // NS
CL = NN // (NC * NS)
CHUNK = 128
N_CHUNKS = 79
EPT = N_CHUNKS * CHUNK
DUMMY = 10224


def _rsqrt_newton(d):
    yi = jnp.int32(0x5F3759DF) - lax.shift_right_logical(
        lax.bitcast_convert_type(d, jnp.int32), 1)
    y = lax.bitcast_convert_type(yi, jnp.float32)
    for _ in range(3):
        y = y * (1.5 - 0.5 * d * y * y)
    return y


def _sc_body(src_hbm, dst_hbm, c_hbm,
             dsti, srci, ones_v, g_v, dis_v, buf_a, buf_b,
             deg_sh, s_sh, dis_sh):
    t = lax.axis_index("s")
    cc = lax.axis_index("c")
    w = cc * NS + t

    def zero_body(i, _):
        buf_a[pl.ds(i * LANES, LANES)] = jnp.zeros((LANES,), jnp.float32)
        return 0
    lax.fori_loop(0, SL // LANES, zero_body, 0)
    pltpu.sync_copy(buf_a, deg_sh.at[pl.ds(t * SL, SL)])
    pltpu.sync_copy(buf_a, s_sh.at[pl.ds(t * SL, SL)])
    for k in range(CHUNK // LANES):
        ones_v[pl.ds(k * LANES, LANES)] = jnp.ones((LANES,), jnp.float32)
    pltpu.sync_copy(dst_hbm.at[t], dsti)
    pltpu.sync_copy(src_hbm.at[t], srci)
    plsc.subcore_barrier()

    def hist_body(j, _):
        pltpu.sync_copy(ones_v, deg_sh.at[dsti.at[j]], add=True)
        return 0
    lax.fori_loop(0, N_CHUNKS, hist_body, 0)
    plsc.subcore_barrier()

    pltpu.sync_copy(deg_sh.at[pl.ds(t * SL, SL)], buf_a)
    def rsqrt_body(i, _):
        d = buf_a[pl.ds(i * LANES, LANES)] + 1.0
        buf_b[pl.ds(i * LANES, LANES)] = _rsqrt_newton(d)
        return 0
    lax.fori_loop(0, SL // LANES, rsqrt_body, 0)
    pltpu.sync_copy(buf_b, dis_sh.at[pl.ds(t * SL, SL)])
    plsc.subcore_barrier()
    pltpu.sync_copy(dis_sh, dis_v)

    def gather_body(j, _):
        for k in range(CHUNK // LANES):
            idx = dsti[j, pl.ds(k * LANES, LANES)]
            g_v[j, pl.ds(k * LANES, LANES)] = plsc.load_gather(dis_v, [idx])
        return 0
    lax.fori_loop(0, N_CHUNKS, gather_body, 0)
    def scat_body(j, _):
        pltpu.sync_copy(g_v.at[j], s_sh.at[srci.at[j]], add=True)
        return 0
    lax.fori_loop(0, N_CHUNKS, scat_body, 0)
    plsc.subcore_barrier()

    pltpu.sync_copy(s_sh.at[pl.ds(w * CL, CL)], buf_a.at[pl.ds(0, CL)])
    def c_body(i, _):
        s = buf_a[pl.ds(i * LANES, LANES)]
        dd = dis_v[pl.ds(w * CL + i * LANES, LANES)]
        ids = w * CL + i * LANES + lax.iota(jnp.int32, 16)
        c = jnp.where(ids < N_NODES, dd * (s + dd), 0.0)
        buf_b[pl.ds(i * LANES, LANES)] = c
        return 0
    lax.fori_loop(0, CL // LANES, c_body, 0)
    pltpu.sync_copy(buf_b.at[pl.ds(0, CL)], c_hbm.at[pl.ds(w * CL, CL)])


_sc_weights = functools.partial(
    pl.kernel,
    out_type=jax.ShapeDtypeStruct((NN,), jnp.float32),
    mesh=plsc.VectorSubcoreMesh(core_axis_name="c", subcore_axis_name="s",
                                num_cores=NC, num_subcores=NS),
    compiler_params=pltpu.CompilerParams(needs_layout_passes=False),
    scratch_types=[
        pltpu.VMEM((N_CHUNKS, CHUNK), jnp.int32),
        pltpu.VMEM((N_CHUNKS, CHUNK), jnp.int32),
        pltpu.VMEM((CHUNK,), jnp.float32),
        pltpu.VMEM((N_CHUNKS, CHUNK), jnp.float32),
        pltpu.VMEM((NN,), jnp.float32),
        pltpu.VMEM((SL,), jnp.float32),
        pltpu.VMEM((SL,), jnp.float32),
        pltpu.VMEM_SHARED((NN,), jnp.float32),
        pltpu.VMEM_SHARED((NN,), jnp.float32),
        pltpu.VMEM_SHARED((NN,), jnp.float32),
    ],
)(_sc_body)


BN = 400


def _tc_body(c_ref, x_ref, w_ref, b_ref, o_ref, acc_ref):
    i = pl.program_id(0)

    @pl.when(i == 0)
    def _init():
        acc_ref[...] = jnp.zeros_like(acc_ref)

    acc_ref[...] += jnp.sum(x_ref[...] * c_ref[...], axis=0, keepdims=True)

    @pl.when(i == pl.num_programs(0) - 1)
    def _fin():
        v = acc_ref[...] * jnp.float32(1.0 / N_NODES)
        o_ref[...] = jnp.dot(v, w_ref[...],
                             preferred_element_type=jnp.float32) + b_ref[...]


def kernel(x, edge_index, W, b):
    pad = EPT * NS - N_EDGES
    ei = jnp.concatenate(
        [edge_index, jnp.full((2, pad), DUMMY, jnp.int32)], axis=1)
    ei = ei.reshape(2, NS, N_CHUNKS, CHUNK)
    c = _sc_weights(ei[0], ei[1])

    c2 = c[:N_NODES].reshape(N_NODES, 1)
    out = pl.pallas_call(
        _tc_body,
        grid=(N_NODES // BN,),
        in_specs=[
            pl.BlockSpec((BN, 1), lambda i: (i, 0)),
            pl.BlockSpec((BN, D_IN), lambda i: (i, 0)),
            pl.BlockSpec((D_IN, D_OUT), lambda i: (0, 0)),
            pl.BlockSpec((1, D_OUT), lambda i: (0, 0)),
        ],
        out_specs=pl.BlockSpec((1, D_OUT), lambda i: (0, 0)),
        out_shape=jax.ShapeDtypeStruct((1, D_OUT), jnp.float32),
        scratch_shapes=[pltpu.VMEM((1, D_IN), jnp.float32)],
    )(c2, x, W, b.reshape(1, D_OUT))
    return out.reshape(D_OUT)


# speedup vs baseline: 103.3778x; 1.3364x over previous
import functools

import jax
import jax.numpy as jnp
from jax import lax
from jax.experimental import pallas as pl
from jax.experimental.pallas import tpu as pltpu
from jax.experimental.pallas import tpu_sc as plsc

N_NODES = 10000
N_EDGES = 160000
D_IN = 256
D_OUT = 512

NC = 2
NS = 16
LANES = 16

NN = 10240
SL = NN // NS
CL = NN // (NC * NS)
CHUNK = 128
N_CHUNKS = 80
GRP = 8
EPT = N_CHUNKS * CHUNK
DUMMY = 10224


def _rsqrt_newton(d):
    yi = jnp.int32(0x5F3759DF) - lax.shift_right_logical(
        lax.bitcast_convert_type(d, jnp.int32), 1)
    y = lax.bitcast_convert_type(yi, jnp.float32)
    for _ in range(3):
        y = y * (1.5 - 0.5 * d * y * y)
    return y


def _sc_body(src_hbm, dst_hbm, c_hbm,
             dsti, srci, ones_v, g_v, dis_v, buf_a, buf_b,
             deg_sh, s_sh, dis_sh, sem):
    t = lax.axis_index("s")
    cc = lax.axis_index("c")
    w = cc * NS + t

    d_dst = pltpu.async_copy(dst_hbm.at[t], dsti, sem)
    d_src = pltpu.async_copy(src_hbm.at[t], srci, sem)
    def zero_body(i, _):
        buf_a[pl.ds(i * LANES, LANES)] = jnp.zeros((LANES,), jnp.float32)
        return 0
    lax.fori_loop(0, SL // LANES, zero_body, 0)
    pltpu.sync_copy(buf_a, deg_sh.at[pl.ds(t * SL, SL)])
    pltpu.sync_copy(buf_a, s_sh.at[pl.ds(t * SL, SL)])
    for k in range(CHUNK // LANES):
        ones_v[pl.ds(k * LANES, LANES)] = jnp.ones((LANES,), jnp.float32)
    d_dst.wait()
    d_src.wait()
    plsc.subcore_barrier()

    def hist_body(g, _):
        descs = [
            pltpu.async_copy(ones_v, deg_sh.at[dsti.at[g * GRP + k]], sem,
                             add=True)
            for k in range(GRP)
        ]
        for d in descs:
            d.wait()
        return 0
    lax.fori_loop(0, N_CHUNKS // GRP, hist_body, 0)
    plsc.subcore_barrier()

    pltpu.sync_copy(deg_sh.at[pl.ds(t * SL, SL)], buf_a)
    def rsqrt_body(i, _):
        d = buf_a[pl.ds(i * LANES, LANES)] + 1.0
        buf_b[pl.ds(i * LANES, LANES)] = _rsqrt_newton(d)
        return 0
    lax.fori_loop(0, SL // LANES, rsqrt_body, 0)
    pltpu.sync_copy(buf_b, dis_sh.at[pl.ds(t * SL, SL)])
    plsc.subcore_barrier()
    pltpu.sync_copy(dis_sh, dis_v)

    def p3_body(g, _):
        for k in range(GRP):
            j = g * GRP + k
            for m in range(CHUNK // LANES):
                idx = dsti[j, pl.ds(m * LANES, LANES)]
                g_v[j, pl.ds(m * LANES, LANES)] = plsc.load_gather(
                    dis_v, [idx])
        descs = [
            pltpu.async_copy(g_v.at[g * GRP + k], s_sh.at[srci.at[g * GRP + k]],
                             sem, add=True)
            for k in range(GRP)
        ]
        for d in descs:
            d.wait()
        return 0
    lax.fori_loop(0, N_CHUNKS // GRP, p3_body, 0)
    plsc.subcore_barrier()

    pltpu.sync_copy(s_sh.at[pl.ds(w * CL, CL)], buf_a.at[pl.ds(0, CL)])
    def c_body(i, _):
        s = buf_a[pl.ds(i * LANES, LANES)]
        dd = dis_v[pl.ds(w * CL + i * LANES, LANES)]
        ids = w * CL + i * LANES + lax.iota(jnp.int32, 16)
        c = jnp.where(ids < N_NODES, dd * (s + dd), 0.0)
        buf_b[pl.ds(i * LANES, LANES)] = c
        return 0
    lax.fori_loop(0, CL // LANES, c_body, 0)
    pltpu.sync_copy(buf_b.at[pl.ds(0, CL)], c_hbm.at[pl.ds(w * CL, CL)])


_sc_weights = functools.partial(
    pl.kernel,
    out_type=jax.ShapeDtypeStruct((NN,), jnp.float32),
    mesh=plsc.VectorSubcoreMesh(core_axis_name="c", subcore_axis_name="s",
                                num_cores=NC, num_subcores=NS),
    compiler_params=pltpu.CompilerParams(needs_layout_passes=False),
    scratch_types=[
        pltpu.VMEM((N_CHUNKS, CHUNK), jnp.int32),
        pltpu.VMEM((N_CHUNKS, CHUNK), jnp.int32),
        pltpu.VMEM((CHUNK,), jnp.float32),
        pltpu.VMEM((N_CHUNKS, CHUNK), jnp.float32),
        pltpu.VMEM((NN,), jnp.float32),
        pltpu.VMEM((SL,), jnp.float32),
        pltpu.VMEM((SL,), jnp.float32),
        pltpu.VMEM_SHARED((NN,), jnp.float32),
        pltpu.VMEM_SHARED((NN,), jnp.float32),
        pltpu.VMEM_SHARED((NN,), jnp.float32),
        pltpu.SemaphoreType.DMA,
    ],
)(_sc_body)


BN = 2000


def _tc_body(c_ref, x_ref, w_ref, b_ref, o_ref, acc_ref):
    i = pl.program_id(0)

    @pl.when(i == 0)
    def _init():
        acc_ref[...] = jnp.zeros_like(acc_ref)

    acc_ref[...] += jnp.sum(x_ref[...] * c_ref[...], axis=0, keepdims=True)

    @pl.when(i == pl.num_programs(0) - 1)
    def _fin():
        v = acc_ref[...] * jnp.float32(1.0 / N_NODES)
        o_ref[...] = jnp.dot(v, w_ref[...],
                             preferred_element_type=jnp.float32) + b_ref[...]


def kernel(x, edge_index, W, b):
    pad = EPT * NS - N_EDGES
    ei = jnp.concatenate(
        [edge_index, jnp.full((2, pad), DUMMY, jnp.int32)], axis=1)
    ei = ei.reshape(2, NS, N_CHUNKS, CHUNK)
    c = _sc_weights(ei[0], ei[1])

    c2 = c[:N_NODES].reshape(N_NODES, 1)
    out = pl.pallas_call(
        _tc_body,
        grid=(N_NODES // BN,),
        in_specs=[
            pl.BlockSpec((BN, 1), lambda i: (i, 0)),
            pl.BlockSpec((BN, D_IN), lambda i: (i, 0)),
            pl.BlockSpec((D_IN, D_OUT), lambda i: (0, 0)),
            pl.BlockSpec((1, D_OUT), lambda i: (0, 0)),
        ],
        out_specs=pl.BlockSpec((1, D_OUT), lambda i: (0, 0)),
        out_shape=jax.ShapeDtypeStruct((1, D_OUT), jnp.float32),
        scratch_shapes=[pltpu.VMEM((1, D_IN), jnp.float32)],
    )(c2, x, W, b.reshape(1, D_OUT))
    return out.reshape(D_OUT)


# speedup vs baseline: 122.6859x; 1.1868x over previous
import functools

import jax
import jax.numpy as jnp
from jax import lax
from jax.experimental import pallas as pl
from jax.experimental.pallas import tpu as pltpu
from jax.experimental.pallas import tpu_sc as plsc

N_NODES = 10000
N_EDGES = 160000
D_IN = 256
D_OUT = 512

NC = 2
NS = 16
LANES = 16

NN = 10240
SL = NN // NS
CL = NN // (NC * NS)
CHUNK = 128
ROWS = N_EDGES // CHUNK
CPT = ROWS // NS
N_EXTRA = ROWS - CPT * NS
GRP = 13
N_GRP = CPT // GRP


def _rsqrt_newton(d):
    yi = jnp.int32(0x5F3759DF) - lax.shift_right_logical(
        lax.bitcast_convert_type(d, jnp.int32), 1)
    y = lax.bitcast_convert_type(yi, jnp.float32)
    for _ in range(3):
        y = y * (1.5 - 0.5 * d * y * y)
    return y


def _sc_body(e_hbm, c_hbm,
             dsti, srci, ones_v, g_v, dis_v, buf_a, buf_b,
             deg_sh, s_sh, dis_sh, sem):
    t = lax.axis_index("s")
    cc = lax.axis_index("c")
    w = cc * NS + t
    has_extra = t < N_EXTRA

    d_dst = pltpu.async_copy(e_hbm.at[1, pl.ds(t * CPT, CPT)],
                             dsti.at[pl.ds(0, CPT)], sem)
    d_src = pltpu.async_copy(e_hbm.at[0, pl.ds(t * CPT, CPT)],
                             srci.at[pl.ds(0, CPT)], sem)

    @pl.when(has_extra)
    def _stage_extra():
        pltpu.sync_copy(e_hbm.at[1, pl.ds(NS * CPT + t, 1)],
                        dsti.at[pl.ds(CPT, 1)])
        pltpu.sync_copy(e_hbm.at[0, pl.ds(NS * CPT + t, 1)],
                        srci.at[pl.ds(CPT, 1)])

    def zero_body(i, _):
        buf_a[pl.ds(i * LANES, LANES)] = jnp.zeros((LANES,), jnp.float32)
        return 0
    lax.fori_loop(0, SL // LANES, zero_body, 0)
    pltpu.sync_copy(buf_a, deg_sh.at[pl.ds(t * SL, SL)])
    pltpu.sync_copy(buf_a, s_sh.at[pl.ds(t * SL, SL)])
    for k in range(CHUNK // LANES):
        ones_v[pl.ds(k * LANES, LANES)] = jnp.ones((LANES,), jnp.float32)
    d_dst.wait()
    d_src.wait()
    plsc.subcore_barrier()

    def hist_body(g, _):
        descs = [
            pltpu.async_copy(ones_v, deg_sh.at[dsti.at[g * GRP + k, 0]], sem,
                             add=True)
            for k in range(GRP)
        ]
        for d in descs:
            d.wait()
        return 0
    lax.fori_loop(0, N_GRP, hist_body, 0)

    @pl.when(has_extra)
    def _hist_extra():
        pltpu.sync_copy(ones_v, deg_sh.at[dsti.at[CPT, 0]], add=True)
    plsc.subcore_barrier()

    pltpu.sync_copy(deg_sh.at[pl.ds(t * SL, SL)], buf_a)
    def rsqrt_body(i, _):
        d = buf_a[pl.ds(i * LANES, LANES)] + 1.0
        buf_b[pl.ds(i * LANES, LANES)] = _rsqrt_newton(d)
        return 0
    lax.fori_loop(0, SL // LANES, rsqrt_body, 0)
    pltpu.sync_copy(buf_b, dis_sh.at[pl.ds(t * SL, SL)])
    plsc.subcore_barrier()
    pltpu.sync_copy(dis_sh, dis_v)

    def p3_body(g, _):
        for k in range(GRP):
            j = g * GRP + k
            for m in range(CHUNK // LANES):
                idx = dsti[j, 0, pl.ds(m * LANES, LANES)]
                g_v[j, 0, pl.ds(m * LANES, LANES)] = plsc.load_gather(
                    dis_v, [idx])
        descs = [
            pltpu.async_copy(g_v.at[g * GRP + k, 0],
                             s_sh.at[srci.at[g * GRP + k, 0]],
                             sem, add=True)
            for k in range(GRP)
        ]
        for d in descs:
            d.wait()
        return 0
    lax.fori_loop(0, N_GRP, p3_body, 0)

    @pl.when(has_extra)
    def _p3_extra():
        for m in range(CHUNK // LANES):
            idx = dsti[CPT, 0, pl.ds(m * LANES, LANES)]
            g_v[CPT, 0, pl.ds(m * LANES, LANES)] = plsc.load_gather(dis_v, [idx])
        pltpu.sync_copy(g_v.at[CPT, 0], s_sh.at[srci.at[CPT, 0]], add=True)
    plsc.subcore_barrier()

    pltpu.sync_copy(s_sh.at[pl.ds(w * CL, CL)], buf_a.at[pl.ds(0, CL)])
    def c_body(i, _):
        s = buf_a[pl.ds(i * LANES, LANES)]
        dd = dis_v[pl.ds(w * CL + i * LANES, LANES)]
        ids = w * CL + i * LANES + lax.iota(jnp.int32, 16)
        c = jnp.where(ids < N_NODES, dd * (s + dd), 0.0)
        buf_b[pl.ds(i * LANES, LANES)] = c
        return 0
    lax.fori_loop(0, CL // LANES, c_body, 0)
    pltpu.sync_copy(buf_b.at[pl.ds(0, CL)], c_hbm.at[pl.ds(w * CL, CL)])


_sc_weights = functools.partial(
    pl.kernel,
    out_type=jax.ShapeDtypeStruct((NN,), jnp.float32),
    mesh=plsc.VectorSubcoreMesh(core_axis_name="c", subcore_axis_name="s",
                                num_cores=NC, num_subcores=NS),
    compiler_params=pltpu.CompilerParams(needs_layout_passes=False),
    scratch_types=[
        pltpu.VMEM((CPT + 1, 1, CHUNK), jnp.int32),
        pltpu.VMEM((CPT + 1, 1, CHUNK), jnp.int32),
        pltpu.VMEM((CHUNK,), jnp.float32),
        pltpu.VMEM((CPT + 1, 1, CHUNK), jnp.float32),
        pltpu.VMEM((NN,), jnp.float32),
        pltpu.VMEM((SL,), jnp.float32),
        pltpu.VMEM((SL,), jnp.float32),
        pltpu.VMEM_SHARED((NN,), jnp.float32),
        pltpu.VMEM_SHARED((NN,), jnp.float32),
        pltpu.VMEM_SHARED((NN,), jnp.float32),
        pltpu.SemaphoreType.DMA,
    ],
)(_sc_body)


BN = 2000


def _tc_body(c_ref, x_ref, w_ref, b_ref, o_ref, acc_ref):
    i = pl.program_id(0)

    @pl.when(i == 0)
    def _init():
        acc_ref[...] = jnp.zeros_like(acc_ref)

    acc_ref[...] += jnp.sum(x_ref[...] * c_ref[...], axis=0, keepdims=True)

    @pl.when(i == pl.num_programs(0) - 1)
    def _fin():
        v = acc_ref[...] * jnp.float32(1.0 / N_NODES)
        o_ref[...] = jnp.dot(v, w_ref[...],
                             preferred_element_type=jnp.float32) + b_ref[...]


def kernel(x, edge_index, W, b):
    ei = edge_index.reshape(2, ROWS, 1, CHUNK)
    c = _sc_weights(ei)

    c2 = c.reshape(NN, 1)
    out = pl.pallas_call(
        _tc_body,
        grid=(N_NODES // BN,),
        in_specs=[
            pl.BlockSpec((BN, 1), lambda i: (i, 0)),
            pl.BlockSpec((BN, D_IN), lambda i: (i, 0)),
            pl.BlockSpec((D_IN, D_OUT), lambda i: (0, 0)),
            pl.BlockSpec((1, D_OUT), lambda i: (0, 0)),
        ],
        out_specs=pl.BlockSpec((1, D_OUT), lambda i: (0, 0)),
        out_shape=jax.ShapeDtypeStruct((1, D_OUT), jnp.float32),
        scratch_shapes=[pltpu.VMEM((1, D_IN), jnp.float32)],
    )(c2, x, W, b.reshape(1, D_OUT))
    return out.reshape(D_OUT)


# speedup vs baseline: 136.3947x; 1.1117x over previous
import functools

import jax
import jax.numpy as jnp
from jax import lax
from jax.experimental import pallas as pl
from jax.experimental.pallas import tpu as pltpu
from jax.experimental.pallas import tpu_sc as plsc

N_NODES = 10000
N_EDGES = 160000
D_IN = 256
D_OUT = 512

NC = 2
NS = 16
LANES = 16

NN = 10240
SL = NN // NS
CL = NN // (NC * NS)
CHUNK = 128
ROWS = N_EDGES // CHUNK
CPT = ROWS // NS
N_EXTRA = ROWS - CPT * NS
EPT = CPT * CHUNK
GRP = 13
N_GRP = CPT // GRP


def _rsqrt_newton(d):
    yi = jnp.int32(0x5F3759DF) - lax.shift_right_logical(
        lax.bitcast_convert_type(d, jnp.int32), 1)
    y = lax.bitcast_convert_type(yi, jnp.float32)
    for _ in range(3):
        y = y * (1.5 - 0.5 * d * y * y)
    return y


def _sc_body(e_hbm, c_hbm,
             dsti, srci, ones_v, g_v, dis_v, buf_a, buf_b,
             deg_sh, s_sh, dis_sh, sem):
    t = lax.axis_index("s")
    cc = lax.axis_index("c")
    w = cc * NS + t
    has_extra = t < N_EXTRA

    d_dst = pltpu.async_copy(e_hbm.at[1, pl.ds(t * EPT, EPT)],
                             dsti.at[pl.ds(0, EPT)], sem)
    d_src = pltpu.async_copy(e_hbm.at[0, pl.ds(t * EPT, EPT)],
                             srci.at[pl.ds(0, EPT)], sem)

    @pl.when(has_extra)
    def _stage_extra():
        pltpu.sync_copy(e_hbm.at[1, pl.ds(NS * EPT + t * CHUNK, CHUNK)],
                        dsti.at[pl.ds(EPT, CHUNK)])
        pltpu.sync_copy(e_hbm.at[0, pl.ds(NS * EPT + t * CHUNK, CHUNK)],
                        srci.at[pl.ds(EPT, CHUNK)])

    def zero_body(i, _):
        buf_a[pl.ds(i * LANES, LANES)] = jnp.zeros((LANES,), jnp.float32)
        return 0
    lax.fori_loop(0, SL // LANES, zero_body, 0)
    pltpu.sync_copy(buf_a, deg_sh.at[pl.ds(t * SL, SL)])
    pltpu.sync_copy(buf_a, s_sh.at[pl.ds(t * SL, SL)])
    for k in range(CHUNK // LANES):
        ones_v[pl.ds(k * LANES, LANES)] = jnp.ones((LANES,), jnp.float32)
    d_dst.wait()
    d_src.wait()
    plsc.subcore_barrier()

    def hist_body(g, _):
        descs = [
            pltpu.async_copy(ones_v, deg_sh.at[dsti.at[pl.ds((g * GRP + k) * CHUNK, CHUNK)]], sem,
                             add=True)
            for k in range(GRP)
        ]
        for d in descs:
            d.wait()
        return 0
    lax.fori_loop(0, N_GRP, hist_body, 0)

    @pl.when(has_extra)
    def _hist_extra():
        pltpu.sync_copy(ones_v, deg_sh.at[dsti.at[pl.ds(EPT, CHUNK)]], add=True)
    plsc.subcore_barrier()

    pltpu.sync_copy(deg_sh.at[pl.ds(t * SL, SL)], buf_a)
    def rsqrt_body(i, _):
        d = buf_a[pl.ds(i * LANES, LANES)] + 1.0
        buf_b[pl.ds(i * LANES, LANES)] = _rsqrt_newton(d)
        return 0
    lax.fori_loop(0, SL // LANES, rsqrt_body, 0)
    pltpu.sync_copy(buf_b, dis_sh.at[pl.ds(t * SL, SL)])
    plsc.subcore_barrier()
    pltpu.sync_copy(dis_sh, dis_v)

    def p3_body(g, _):
        for k in range(GRP):
            j = (g * GRP + k) * CHUNK
            for m in range(CHUNK // LANES):
                idx = dsti[pl.ds(j + m * LANES, LANES)]
                g_v[pl.ds(j + m * LANES, LANES)] = plsc.load_gather(
                    dis_v, [idx])
        descs = [
            pltpu.async_copy(g_v.at[pl.ds((g * GRP + k) * CHUNK, CHUNK)],
                             s_sh.at[srci.at[pl.ds((g * GRP + k) * CHUNK,
                                                   CHUNK)]],
                             sem, add=True)
            for k in range(GRP)
        ]
        for d in descs:
            d.wait()
        return 0
    lax.fori_loop(0, N_GRP, p3_body, 0)

    @pl.when(has_extra)
    def _p3_extra():
        for m in range(CHUNK // LANES):
            idx = dsti[pl.ds(EPT + m * LANES, LANES)]
            g_v[pl.ds(EPT + m * LANES, LANES)] = plsc.load_gather(dis_v, [idx])
        pltpu.sync_copy(g_v.at[pl.ds(EPT, CHUNK)],
                        s_sh.at[srci.at[pl.ds(EPT, CHUNK)]], add=True)
    plsc.subcore_barrier()

    pltpu.sync_copy(s_sh.at[pl.ds(w * CL, CL)], buf_a.at[pl.ds(0, CL)])
    def c_body(i, _):
        s = buf_a[pl.ds(i * LANES, LANES)]
        dd = dis_v[pl.ds(w * CL + i * LANES, LANES)]
        ids = w * CL + i * LANES + lax.iota(jnp.int32, 16)
        c = jnp.where(ids < N_NODES, dd * (s + dd), 0.0)
        buf_b[pl.ds(i * LANES, LANES)] = c
        return 0
    lax.fori_loop(0, CL // LANES, c_body, 0)
    pltpu.sync_copy(buf_b.at[pl.ds(0, CL)], c_hbm.at[pl.ds(w * CL, CL)])


_sc_weights = functools.partial(
    pl.kernel,
    out_type=jax.ShapeDtypeStruct((NN,), jnp.float32),
    mesh=plsc.VectorSubcoreMesh(core_axis_name="c", subcore_axis_name="s",
                                num_cores=NC, num_subcores=NS),
    compiler_params=pltpu.CompilerParams(needs_layout_passes=False),
    scratch_types=[
        pltpu.VMEM((EPT + CHUNK,), jnp.int32),
        pltpu.VMEM((EPT + CHUNK,), jnp.int32),
        pltpu.VMEM((CHUNK,), jnp.float32),
        pltpu.VMEM((EPT + CHUNK,), jnp.float32),
        pltpu.VMEM((NN,), jnp.float32),
        pltpu.VMEM((SL,), jnp.float32),
        pltpu.VMEM((SL,), jnp.float32),
        pltpu.VMEM_SHARED((NN,), jnp.float32),
        pltpu.VMEM_SHARED((NN,), jnp.float32),
        pltpu.VMEM_SHARED((NN,), jnp.float32),
        pltpu.SemaphoreType.DMA,
    ],
)(_sc_body)


BN = 2000


def _tc_body(c_ref, x_ref, w_ref, b_ref, o_ref, acc_ref):
    i = pl.program_id(0)

    @pl.when(i == 0)
    def _init():
        acc_ref[...] = jnp.zeros_like(acc_ref)

    cb = c_ref[pl.ds(i, 1), :]
    acc_ref[...] += jnp.dot(cb, x_ref[...],
                            preferred_element_type=jnp.float32)

    @pl.when(i == pl.num_programs(0) - 1)
    def _fin():
        v = acc_ref[...] * jnp.float32(1.0 / N_NODES)
        o_ref[...] = jnp.dot(v, w_ref[...],
                             preferred_element_type=jnp.float32) + b_ref[...]


def kernel(x, edge_index, W, b):
    c = _sc_weights(edge_index)

    c2 = c[:N_NODES].reshape(N_NODES // BN, BN)
    out = pl.pallas_call(
        _tc_body,
        grid=(N_NODES // BN,),
        in_specs=[
            pl.BlockSpec((N_NODES // BN, BN), lambda i: (0, 0)),
            pl.BlockSpec((BN, D_IN), lambda i: (i, 0)),
            pl.BlockSpec((D_IN, D_OUT), lambda i: (0, 0)),
            pl.BlockSpec((1, D_OUT), lambda i: (0, 0)),
        ],
        out_specs=pl.BlockSpec((1, D_OUT), lambda i: (0, 0)),
        out_shape=jax.ShapeDtypeStruct((1, D_OUT), jnp.float32),
        scratch_shapes=[pltpu.VMEM((1, D_IN), jnp.float32)],
    )(c2, x, W, b.reshape(1, D_OUT))
    return out.reshape(D_OUT)


# speedup vs baseline: 141.9971x; 1.0411x over previous
import functools

import jax
import jax.numpy as jnp
from jax import lax
from jax.experimental import pallas as pl
from jax.experimental.pallas import tpu as pltpu
from jax.experimental.pallas import tpu_sc as plsc

N_NODES = 10000
N_EDGES = 160000
D_IN = 256
D_OUT = 512

NC = 2
NS = 16
LANES = 16

NN = 10240
SL = NN // NS
CL = NN // (NC * NS)
CHUNK = 128
ROWS = N_EDGES // CHUNK
CPT = ROWS // NS
N_EXTRA = ROWS - CPT * NS
EPT = CPT * CHUNK
GRP = 13
N_GRP = CPT // GRP


def _rsqrt_newton(d):
    yi = jnp.int32(0x5F3759DF) - lax.shift_right_logical(
        lax.bitcast_convert_type(d, jnp.int32), 1)
    y = lax.bitcast_convert_type(yi, jnp.float32)
    for _ in range(3):
        y = y * (1.5 - 0.5 * d * y * y)
    return y


def _sc_body(e_hbm, c_hbm,
             dsti, srci, ones_v, g_v, dis_v, buf_a, buf_b,
             deg_sh, s_sh, dis_sh, sem):
    t = lax.axis_index("s")
    cc = lax.axis_index("c")
    w = cc * NS + t
    has_extra = t < N_EXTRA

    d_dst = pltpu.async_copy(e_hbm.at[1, pl.ds(t * EPT, EPT)],
                             dsti.at[pl.ds(0, EPT)], sem)
    d_src = pltpu.async_copy(e_hbm.at[0, pl.ds(t * EPT, EPT)],
                             srci.at[pl.ds(0, EPT)], sem)

    @pl.when(has_extra)
    def _stage_extra():
        pltpu.sync_copy(e_hbm.at[1, pl.ds(NS * EPT + t * CHUNK, CHUNK)],
                        dsti.at[pl.ds(EPT, CHUNK)])
        pltpu.sync_copy(e_hbm.at[0, pl.ds(NS * EPT + t * CHUNK, CHUNK)],
                        srci.at[pl.ds(EPT, CHUNK)])

    def zero_body(i, _):
        buf_a[pl.ds(i * LANES, LANES)] = jnp.zeros((LANES,), jnp.float32)
        return 0
    lax.fori_loop(0, SL // LANES, zero_body, 0)
    pltpu.sync_copy(buf_a, deg_sh.at[pl.ds(t * SL, SL)])
    pltpu.sync_copy(buf_a, s_sh.at[pl.ds(t * SL, SL)])
    for k in range(CHUNK // LANES):
        ones_v[pl.ds(k * LANES, LANES)] = jnp.ones((LANES,), jnp.float32)
    d_dst.wait()
    d_src.wait()
    plsc.subcore_barrier()

    def _fire_hist(g):
        return [
            pltpu.async_copy(
                ones_v, deg_sh.at[dsti.at[pl.ds((g * GRP + k) * CHUNK, CHUNK)]],
                sem, add=True)
            for k in range(GRP)
        ]
    d0 = _fire_hist(0)
    def hist_body(g, _):
        descs = _fire_hist(g)
        for d in descs:
            d.wait()
        return 0
    lax.fori_loop(1, N_GRP, hist_body, 0)
    for d in d0:
        d.wait()

    @pl.when(has_extra)
    def _hist_extra():
        pltpu.sync_copy(ones_v, deg_sh.at[dsti.at[pl.ds(EPT, CHUNK)]], add=True)
    plsc.subcore_barrier()

    pltpu.sync_copy(deg_sh.at[pl.ds(t * SL, SL)], buf_a)
    def rsqrt_body(i, _):
        d = buf_a[pl.ds(i * LANES, LANES)] + 1.0
        buf_b[pl.ds(i * LANES, LANES)] = _rsqrt_newton(d)
        return 0
    lax.fori_loop(0, SL // LANES, rsqrt_body, 0)
    pltpu.sync_copy(buf_b, dis_sh.at[pl.ds(t * SL, SL)])
    plsc.subcore_barrier()
    pltpu.sync_copy(dis_sh, dis_v)

    def _gather_grp(g):
        for k in range(GRP):
            j = (g * GRP + k) * CHUNK
            for m in range(CHUNK // LANES):
                idx = dsti[pl.ds(j + m * LANES, LANES)]
                g_v[pl.ds(j + m * LANES, LANES)] = plsc.load_gather(
                    dis_v, [idx])

    def _fire_p3(g):
        return [
            pltpu.async_copy(g_v.at[pl.ds((g * GRP + k) * CHUNK, CHUNK)],
                             s_sh.at[srci.at[pl.ds((g * GRP + k) * CHUNK,
                                                   CHUNK)]],
                             sem, add=True)
            for k in range(GRP)
        ]
    _gather_grp(0)
    d1 = _fire_p3(0)
    def p3_body(g, _):
        _gather_grp(g)
        descs = _fire_p3(g)
        for d in descs:
            d.wait()
        return 0
    lax.fori_loop(1, N_GRP, p3_body, 0)
    for d in d1:
        d.wait()

    @pl.when(has_extra)
    def _p3_extra():
        for m in range(CHUNK // LANES):
            idx = dsti[pl.ds(EPT + m * LANES, LANES)]
            g_v[pl.ds(EPT + m * LANES, LANES)] = plsc.load_gather(dis_v, [idx])
        pltpu.sync_copy(g_v.at[pl.ds(EPT, CHUNK)],
                        s_sh.at[srci.at[pl.ds(EPT, CHUNK)]], add=True)
    plsc.subcore_barrier()

    pltpu.sync_copy(s_sh.at[pl.ds(w * CL, CL)], buf_a.at[pl.ds(0, CL)])
    def c_body(i, _):
        s = buf_a[pl.ds(i * LANES, LANES)]
        dd = dis_v[pl.ds(w * CL + i * LANES, LANES)]
        ids = w * CL + i * LANES + lax.iota(jnp.int32, 16)
        c = jnp.where(ids < N_NODES, dd * (s + dd), 0.0)
        buf_b[pl.ds(i * LANES, LANES)] = c
        return 0
    lax.fori_loop(0, CL // LANES, c_body, 0)
    pltpu.sync_copy(buf_b.at[pl.ds(0, CL)], c_hbm.at[pl.ds(w * CL, CL)])


_sc_weights = functools.partial(
    pl.kernel,
    out_type=jax.ShapeDtypeStruct((NN,), jnp.float32),
    mesh=plsc.VectorSubcoreMesh(core_axis_name="c", subcore_axis_name="s",
                                num_cores=NC, num_subcores=NS),
    compiler_params=pltpu.CompilerParams(needs_layout_passes=False),
    scratch_types=[
        pltpu.VMEM((EPT + CHUNK,), jnp.int32),
        pltpu.VMEM((EPT + CHUNK,), jnp.int32),
        pltpu.VMEM((CHUNK,), jnp.float32),
        pltpu.VMEM((EPT + CHUNK,), jnp.float32),
        pltpu.VMEM((NN,), jnp.float32),
        pltpu.VMEM((SL,), jnp.float32),
        pltpu.VMEM((SL,), jnp.float32),
        pltpu.VMEM_SHARED((NN,), jnp.float32),
        pltpu.VMEM_SHARED((NN,), jnp.float32),
        pltpu.VMEM_SHARED((NN,), jnp.float32),
        pltpu.SemaphoreType.DMA,
    ],
)(_sc_body)


BN = 2000


def _tc_body(c_ref, x_ref, w_ref, b_ref, o_ref, acc_ref):
    i = pl.program_id(0)

    @pl.when(i == 0)
    def _init():
        acc_ref[...] = jnp.zeros_like(acc_ref)

    cb = c_ref[pl.ds(i, 1), :]
    acc_ref[...] += jnp.dot(cb, x_ref[...],
                            preferred_element_type=jnp.float32)

    @pl.when(i == pl.num_programs(0) - 1)
    def _fin():
        v = acc_ref[...] * jnp.float32(1.0 / N_NODES)
        o_ref[...] = jnp.dot(v, w_ref[...],
                             preferred_element_type=jnp.float32) + b_ref[...]


def kernel(x, edge_index, W, b):
    c = _sc_weights(edge_index)

    c2 = c[:N_NODES].reshape(N_NODES // BN, BN)
    out = pl.pallas_call(
        _tc_body,
        grid=(N_NODES // BN,),
        in_specs=[
            pl.BlockSpec((N_NODES // BN, BN), lambda i: (0, 0)),
            pl.BlockSpec((BN, D_IN), lambda i: (i, 0)),
            pl.BlockSpec((D_IN, D_OUT), lambda i: (0, 0)),
            pl.BlockSpec((1, D_OUT), lambda i: (0, 0)),
        ],
        out_specs=pl.BlockSpec((1, D_OUT), lambda i: (0, 0)),
        out_shape=jax.ShapeDtypeStruct((1, D_OUT), jnp.float32),
        scratch_shapes=[pltpu.VMEM((1, D_IN), jnp.float32)],
    )(c2, x, W, b.reshape(1, D_OUT))
    return out.reshape(D_OUT)
